# baseline (speedup 1.0000x reference)
import dataclasses
import functools
import jax
import jax.numpy as jnp
from jax import lax
from jax.experimental import pallas as pl
from jax.experimental.pallas import tpu as pltpu
from jax.experimental.pallas import tpu_sc as plsc

N = 10000
E = 160000
D = 256
DE = 16

NC = 2
NS = 16
CH = 128
N_PAD = 10240
E_PAD = 163840
N_CHUNKS = E_PAD // CH
CPS = N_CHUNKS // NS
ROWS = N_PAD // NS
GW = 32
BLK = 8

NLOC = N_PAD // NC
EPT = E_PAD // NS
PH = 1024
NPH = EPT // PH
CHN = 64
PAIR = 2 * CHN
CCAP = PH + PAIR + 16
ZDR = 2 * NLOC // NS
SBITS = 14
PAD_PK = N

_mesh = plsc.VectorSubcoreMesh(core_axis_name="c", subcore_axis_name="s",
                               num_cores=NC, num_subcores=NS)

_sc_params = pltpu.CompilerParams()
if "needs_layout_passes" in pltpu.CompilerParams.__dataclass_fields__:
    _sc_params = dataclasses.replace(_sc_params, needs_layout_passes=False)


@functools.partial(
    pl.kernel,
    out_type=jax.ShapeDtypeStruct((NC, N_PAD, 16), jnp.float32),
    mesh=_mesh,
    scratch_types=[
        pltpu.VMEM_SHARED((N_PAD, 16), jnp.float32),
        pltpu.VMEM((CPS // 2, CH), jnp.int32),
        pltpu.VMEM((CH, 16), jnp.float32),
        pltpu.VMEM((CH, 16), jnp.float32),
    ],
)
def _sc_hist(recv_hbm, hist_out, hist_sh, ridx, onehot, zbuf):
    c = lax.axis_index("c")
    s = lax.axis_index("s")
    w = s * NC + c
    zeros16 = jnp.zeros((16,), jnp.float32)
    one0 = jnp.where(lax.iota(jnp.int32, 16) == 0,
                     jnp.float32(1.0), jnp.float32(0.0))

    @pl.loop(0, CH)
    def _(i):
        onehot[i, :] = one0
        zbuf[i, :] = zeros16

    @pl.loop(0, ROWS // CH)
    def _(k):
        pltpu.sync_copy(zbuf, hist_sh.at[pl.ds(s * ROWS + k * CH, CH)])

    pltpu.sync_copy(recv_hbm.at[pl.ds(w * (CPS // 2), CPS // 2)], ridx)
    plsc.subcore_barrier()

    @pl.loop(0, CPS // 2)
    def _(j):
        pltpu.sync_copy(onehot, hist_sh.at[ridx.at[j]], add=True)

    plsc.subcore_barrier()
    pltpu.sync_copy(hist_sh.at[pl.ds(s * ROWS, ROWS)],
                    hist_out.at[c].at[pl.ds(s * ROWS, ROWS)])


@functools.partial(
    pl.kernel,
    out_type=jax.ShapeDtypeStruct((2 * N_PAD, D // 2), jnp.float32),
    mesh=_mesh,
    scratch_types=[
        pltpu.VMEM_SHARED((2 * NLOC, D // 2), jnp.float32),
        pltpu.VMEM((PH,), jnp.int32),
        pltpu.VMEM((PH,), jnp.int32),
        pltpu.VMEM((CCAP,), jnp.int32),
        pltpu.VMEM((2, CHN), jnp.int32),
        pltpu.VMEM((2, PAIR), jnp.int32),
        pltpu.VMEM((CHN, 2, D // 2), jnp.float32),
        pltpu.VMEM((CHN, 2, D // 2), jnp.float32),
        pltpu.SemaphoreType.DMA,
        pltpu.SemaphoreType.DMA,
        pltpu.SemaphoreType.DMA,
        pltpu.SemaphoreType.DMA,
    ],
    compiler_params=_sc_params,
)
def _sc_edges(hs_hbm, send_hbm, recv_hbm, acc_out, acc_sh,
              rawS, rawR, cpk, sstage, rstage, hbufA, hbufB,
              gsA, gsB, ssA, ssB):
    c = lax.axis_index("c")
    s = lax.axis_index("s")
    base = c * NLOC
    zeros16 = jnp.zeros((16,), jnp.float32)
    tru16 = jnp.ones((16,), jnp.bool_)
    mask_s = jnp.int32((1 << SBITS) - 1)
    iota16 = lax.iota(jnp.int32, 16)
    hbufs = (hbufA, hbufB)
    gsems = (gsA, gsB)
    ssems = (ssA, ssB)

    @pl.loop(0, CHN)
    def _(i):
        for h in range(2):
            @pl.loop(0, D // 2, step=16)
            def _(q):
                hbufA[i, h, pl.ds(q, 16)] = zeros16

    @pl.loop(0, ZDR // PAIR)
    def _(k):
        pltpu.sync_copy(hbufA.reshape(PAIR, D // 2),
                        acc_sh.at[pl.ds(s * ZDR + k * PAIR, PAIR)])

    plsc.subcore_barrier()

    ebase = s * EPT

    @pl.loop(0, NPH)
    def _(ph):
        pltpu.sync_copy(send_hbm.at[pl.ds(ebase + ph * PH, PH)], rawS)
        pltpu.sync_copy(recv_hbm.at[pl.ds(ebase + ph * PH, PH)], rawR)

        def scan_body(g, cnt):
            off = g * 16
            sv = rawS[pl.ds(off, 16)]
            rv = rawR[pl.ds(off, 16)]
            rloc = rv - base
            m = jnp.logical_and(rloc >= 0, rloc < NLOC)
            packed = jnp.bitwise_or(sv, lax.shift_left(rloc, SBITS))
            inc = plsc.cumsum(m.astype(jnp.int32))
            plsc.store_scatter(cpk, [cnt + inc - 1], packed, mask=m)
            return cnt + jnp.sum(m.astype(jnp.int32))

        cnt = lax.fori_loop(0, PH // 16, scan_body, jnp.int32(0))

        pad_pk = jnp.full((16,), PAD_PK, jnp.int32)
        for t in range(PAIR // 16):
            plsc.store_scatter(cpk, [cnt + iota16 + t * 16], pad_pk)
        npairs = (cnt + PAIR - 1) // PAIR

        def pair_body(k, carry):
            o = k * PAIR
            pA = jnp.bitwise_and(k, 1)
            pB = 1 - pA

            @pl.when(k > 0)
            def _():
                pltpu.make_async_copy(
                    hbufA.reshape(PAIR, D // 2),
                    acc_sh.at[rstage.at[pA]], ssA).wait()
                pltpu.make_async_copy(
                    hbufB.reshape(PAIR, D // 2),
                    acc_sh.at[rstage.at[pB]], ssB).wait()

            for q, pr in ((0, pA), (1, pB)):
                pvec = jnp.full((16,), 0, jnp.int32) + pr
                for t in range(CHN // 16):
                    w = cpk[pl.ds(o + q * CHN + t * 16, 16)]
                    sstage[pr, pl.ds(t * 16, 16)] = jnp.bitwise_and(w, mask_s)
                    rloc2 = lax.shift_right_logical(w, SBITS) * 2
                    pos = iota16 * 2 + t * 32
                    plsc.store_scatter(rstage, [pvec, pos], rloc2)
                    plsc.store_scatter(rstage, [pvec, pos + 1], rloc2 + 1)
            gA = pltpu.async_copy(hs_hbm.at[sstage.at[pA]], hbufA, gsA)
            gB = pltpu.async_copy(hs_hbm.at[sstage.at[pB]], hbufB, gsB)
            gA.wait()
            pltpu.async_copy(hbufA.reshape(PAIR, D // 2),
                             acc_sh.at[rstage.at[pA]], ssA, add=True)
            gB.wait()
            pltpu.async_copy(hbufB.reshape(PAIR, D // 2),
                             acc_sh.at[rstage.at[pB]], ssB, add=True)
            return carry

        lax.fori_loop(0, npairs, pair_body, jnp.int32(0))

        @pl.when(npairs > 0)
        def _():
            pL = jnp.bitwise_and(npairs - 1, 1)
            pltpu.make_async_copy(hbufA.reshape(PAIR, D // 2),
                                  acc_sh.at[rstage.at[pL]], ssA).wait()
            pltpu.make_async_copy(hbufB.reshape(PAIR, D // 2),
                                  acc_sh.at[rstage.at[1 - pL]], ssB).wait()

    plsc.subcore_barrier()
    pltpu.sync_copy(acc_sh.at[pl.ds(s * ZDR, ZDR)],
                    acc_out.at[pl.ds(2 * base + s * ZDR, ZDR)])


CPW = N_CHUNKS // (NC * NS)


@functools.partial(
    pl.kernel,
    out_type=jax.ShapeDtypeStruct((NC, N_PAD, GW), jnp.float32),
    mesh=_mesh,
    scratch_types=[
        pltpu.VMEM_SHARED((N_PAD, GW), jnp.float32),
        pltpu.VMEM((BLK, CH), jnp.int32),
        pltpu.VMEM((BLK, CH), jnp.int32),
        pltpu.VMEM((2, CH), jnp.float32),
        pltpu.VMEM((2, DE, CH), jnp.float32),
        pltpu.VMEM((CH, GW), jnp.float32),
        pltpu.VMEM((CH, GW), jnp.float32),
        pltpu.SemaphoreType.DMA,
        pltpu.SemaphoreType.DMA,
        pltpu.SemaphoreType.DMA,
        pltpu.SemaphoreType.DMA,
    ],
    compiler_params=_sc_params,
)
def _sc_gfeat(send_hbm, recv_hbm, eft_hbm, rs_hbm,
              g_out, g_sh, sidx, ridx, rsbuf, eftv, gbufA, gbufB,
              leA, leB, ssA, ssB):
    c = lax.axis_index("c")
    s = lax.axis_index("s")
    w = s * NC + c
    zeros16 = jnp.zeros((16,), jnp.float32)
    iota16 = lax.iota(jnp.int32, 16)
    gbufs = (gbufA, gbufB)
    lsems = (leA, leB)
    ssems = (ssA, ssB)

    @pl.loop(0, CH)
    def _(i):
        @pl.loop(0, GW, step=16)
        def _(q):
            gbufA[i, pl.ds(q, 16)] = zeros16
            gbufB[i, pl.ds(q, 16)] = zeros16

    @pl.loop(0, ROWS // CH)
    def _(k):
        pltpu.sync_copy(gbufA, g_sh.at[pl.ds(s * ROWS + k * CH, CH)])

    plsc.subcore_barrier()

    @pl.loop(0, CPW // BLK)
    def _(b):
        base = w * CPW + b * BLK
        pltpu.sync_copy(send_hbm.at[pl.ds(base, BLK)], sidx)
        pltpu.sync_copy(recv_hbm.at[pl.ds(base, BLK)], ridx)

        def load(j, p):
            e = pltpu.async_copy(
                eft_hbm.at[:, pl.ds((base + j) * CH, CH)], eftv.at[p],
                lsems[p])
            r = pltpu.async_copy(rs_hbm.at[sidx.at[j]], rsbuf.at[p],
                                 lsems[p])
            return e, r

        ld = {0: load(0, 0), 1: load(1, 1)}
        sc = {}
        for j in range(BLK):
            p = j & 1
            ld[j][0].wait()
            ld[j][1].wait()
            if j >= 2:
                sc[j - 2].wait()
            for g in range(CH // 16):
                rg = rsbuf[p, pl.ds(g * 16, 16)]
                rows = iota16 + g * 16
                for d in range(DE):
                    v = eftv[p, d, pl.ds(g * 16, 16)] * rg
                    plsc.store_scatter(
                        gbufs[p], [rows, jnp.full((16,), d, jnp.int32)], v)
                plsc.store_scatter(
                    gbufs[p], [rows, jnp.full((16,), DE, jnp.int32)], rg)
            sc[j] = pltpu.async_copy(
                gbufs[p], g_sh.at[ridx.at[j]], ssems[p], add=True)
            if j + 2 < BLK:
                ld[j + 2] = load(j + 2, p)
        sc[BLK - 2].wait()
        sc[BLK - 1].wait()

    plsc.subcore_barrier()
    pltpu.sync_copy(g_sh.at[pl.ds(s * ROWS, ROWS)],
                    g_out.at[c].at[pl.ds(s * ROWS, ROWS)])


def _tc_h_body(x_ref, w_ref, b_ref, h_ref):
    h_ref[...] = (
        jax.lax.dot_general(x_ref[...], w_ref[...], (((1,), (0,)), ((), ())),
                            precision=jax.lax.Precision.HIGHEST,
                            preferred_element_type=jnp.float32)
        + b_ref[...])


def _tc_scale_body(h_ref, hist_ref, hs_ref, rs_ref):
    i = pl.program_id(0)
    deg = jnp.sum(hist_ref[...], axis=(0, 2)) + 1.0
    rs = lax.rsqrt(deg)
    rows = i * h_ref.shape[0] + lax.broadcasted_iota(
        jnp.int32, (h_ref.shape[0], 1), 0)
    hs_ref[...] = jnp.where(rows < N, h_ref[...] * rs[:, None], 0.0)
    rs_ref[...] = rs


def _tc_final_body(acc_ref, g_ref, hs_ref, rs_ref, we_ref, be_ref, out_ref):
    g = g_ref[0, ...] + g_ref[1, ...]
    ef_part = jax.lax.dot_general(
        g[:, :DE], we_ref[...], (((1,), (0,)), ((), ())),
        precision=jax.lax.Precision.HIGHEST,
        preferred_element_type=jnp.float32)
    bias_part = g[:, DE:DE + 1] * be_ref[...][None, :]
    out_ref[...] = rs_ref[...][:, None] * (
        hs_ref[...] + acc_ref[...] + ef_part + bias_part)


@jax.jit
def kernel(node_features, senders, receivers, edge_features,
           W_kernel, W_bias, We_kernel, We_bias):
    senders = senders.astype(jnp.int32)
    receivers = receivers.astype(jnp.int32)

    pad_e = E_PAD - E
    s_flat = jnp.concatenate([senders, jnp.full((pad_e,), N, jnp.int32)])
    r_flat = jnp.concatenate([receivers, jnp.full((pad_e,), N, jnp.int32)])
    s2d = s_flat.reshape(N_CHUNKS, CH)
    r2d = r_flat.reshape(N_CHUNKS, CH)
    eft = jnp.concatenate(
        [edge_features.T, jnp.zeros((DE, pad_e), jnp.float32)], axis=1)
    xpad = jnp.concatenate(
        [node_features, jnp.zeros((N_PAD - N, D), jnp.float32)])

    hist = _sc_hist(r2d)

    RB = 1024
    grid = (N_PAD // RB,)

    h = pl.pallas_call(
        _tc_h_body,
        grid=grid,
        in_specs=[pl.BlockSpec((RB, D), lambda i: (i, 0)),
                  pl.BlockSpec((D, D), lambda i: (0, 0)),
                  pl.BlockSpec((D,), lambda i: (0,))],
        out_specs=pl.BlockSpec((RB, D), lambda i: (i, 0)),
        out_shape=jax.ShapeDtypeStruct((N_PAD, D), jnp.float32),
    )(xpad, W_kernel, W_bias)

    hs, rs = pl.pallas_call(
        _tc_scale_body,
        grid=grid,
        in_specs=[pl.BlockSpec((RB, D), lambda i: (i, 0)),
                  pl.BlockSpec((NC, RB, 16), lambda i: (0, i, 0))],
        out_specs=(pl.BlockSpec((RB, D), lambda i: (i, 0)),
                   pl.BlockSpec((RB,), lambda i: (i,))),
        out_shape=(jax.ShapeDtypeStruct((N_PAD, D), jnp.float32),
                   jax.ShapeDtypeStruct((N_PAD,), jnp.float32)),
    )(h, hist)

    acc = _sc_edges(hs.reshape(N_PAD, 2, D // 2),
                    s_flat, r_flat).reshape(N_PAD, D)
    g2 = _sc_gfeat(s2d, r2d, eft, rs)

    out = pl.pallas_call(
        _tc_final_body,
        grid=grid,
        in_specs=[pl.BlockSpec((RB, D), lambda i: (i, 0)),
                  pl.BlockSpec((NC, RB, GW), lambda i: (0, i, 0)),
                  pl.BlockSpec((RB, D), lambda i: (i, 0)),
                  pl.BlockSpec((RB,), lambda i: (i,)),
                  pl.BlockSpec((DE, D), lambda i: (0, 0)),
                  pl.BlockSpec((D,), lambda i: (0,))],
        out_specs=pl.BlockSpec((RB, D), lambda i: (i, 0)),
        out_shape=jax.ShapeDtypeStruct((N_PAD, D), jnp.float32),
    )(acc, g2, hs, rs, We_kernel, We_bias)

    return out[:N]


# speedup vs baseline: 1.0125x; 1.0125x over previous
import dataclasses
import functools
import jax
import jax.numpy as jnp
from jax import lax
from jax.experimental import pallas as pl
from jax.experimental.pallas import tpu as pltpu
from jax.experimental.pallas import tpu_sc as plsc

N = 10000
E = 160000
D = 256
DE = 16

NC = 2
NS = 16
CH = 128
N_PAD = 10240
E_PAD = 163840
N_CHUNKS = E_PAD // CH
CPS = N_CHUNKS // NS
ROWS = N_PAD // NS
GW = 32
BLK = 8

NLOC = N_PAD // NC
EPT = E_PAD // NS
PH = 1024
NPH = EPT // PH
CHN = 64
PAIR = 2 * CHN
CCAP = PH + PAIR + 16
ZDR = 2 * NLOC // NS
SBITS = 14
PAD_PK = N

_mesh = plsc.VectorSubcoreMesh(core_axis_name="c", subcore_axis_name="s",
                               num_cores=NC, num_subcores=NS)

_sc_params = pltpu.CompilerParams()
if "needs_layout_passes" in pltpu.CompilerParams.__dataclass_fields__:
    _sc_params = dataclasses.replace(_sc_params, needs_layout_passes=False)


@functools.partial(
    pl.kernel,
    out_type=jax.ShapeDtypeStruct((NC, N_PAD, 16), jnp.float32),
    mesh=_mesh,
    scratch_types=[
        pltpu.VMEM_SHARED((N_PAD, 16), jnp.float32),
        pltpu.VMEM((CPS // 2, CH), jnp.int32),
        pltpu.VMEM((CH, 16), jnp.float32),
        pltpu.VMEM((CH, 16), jnp.float32),
    ],
)
def _sc_hist(recv_hbm, hist_out, hist_sh, ridx, onehot, zbuf):
    c = lax.axis_index("c")
    s = lax.axis_index("s")
    w = s * NC + c
    zeros16 = jnp.zeros((16,), jnp.float32)
    one0 = jnp.where(lax.iota(jnp.int32, 16) == 0,
                     jnp.float32(1.0), jnp.float32(0.0))

    @pl.loop(0, CH)
    def _(i):
        onehot[i, :] = one0
        zbuf[i, :] = zeros16

    @pl.loop(0, ROWS // CH)
    def _(k):
        pltpu.sync_copy(zbuf, hist_sh.at[pl.ds(s * ROWS + k * CH, CH)])

    pltpu.sync_copy(recv_hbm.at[pl.ds(w * (CPS // 2), CPS // 2)], ridx)
    plsc.subcore_barrier()

    @pl.loop(0, CPS // 2)
    def _(j):
        pltpu.sync_copy(onehot, hist_sh.at[ridx.at[j]], add=True)

    plsc.subcore_barrier()
    pltpu.sync_copy(hist_sh.at[pl.ds(s * ROWS, ROWS)],
                    hist_out.at[c].at[pl.ds(s * ROWS, ROWS)])


@functools.partial(
    pl.kernel,
    out_type=jax.ShapeDtypeStruct((N_PAD, 2, D // 2), jnp.float32),
    mesh=_mesh,
    scratch_types=[
        pltpu.VMEM_SHARED((NLOC, 2, D // 2), jnp.float32),
        pltpu.VMEM((PH,), jnp.int32),
        pltpu.VMEM((PH,), jnp.int32),
        pltpu.VMEM((CCAP,), jnp.int32),
        pltpu.VMEM((2, CHN), jnp.int32),
        pltpu.VMEM((2, CHN), jnp.int32),
        pltpu.VMEM((CHN, 2, D // 2), jnp.float32),
        pltpu.VMEM((CHN, 2, D // 2), jnp.float32),
        pltpu.SemaphoreType.DMA,
        pltpu.SemaphoreType.DMA,
        pltpu.SemaphoreType.DMA,
        pltpu.SemaphoreType.DMA,
    ],
    compiler_params=_sc_params,
)
def _sc_edges(hs_hbm, send_hbm, recv_hbm, acc_out, acc_sh,
              rawS, rawR, cpk, sstage, rstage, hbufA, hbufB,
              gsA, gsB, ssA, ssB):
    c = lax.axis_index("c")
    s = lax.axis_index("s")
    base = c * NLOC
    zeros16 = jnp.zeros((16,), jnp.float32)
    tru16 = jnp.ones((16,), jnp.bool_)
    mask_s = jnp.int32((1 << SBITS) - 1)
    iota16 = lax.iota(jnp.int32, 16)
    hbufs = (hbufA, hbufB)
    gsems = (gsA, gsB)
    ssems = (ssA, ssB)

    @pl.loop(0, CHN)
    def _(i):
        for h in range(2):
            @pl.loop(0, D // 2, step=16)
            def _(q):
                hbufA[i, h, pl.ds(q, 16)] = zeros16

    @pl.loop(0, NLOC // NS // CHN)
    def _(k):
        pltpu.sync_copy(
            hbufA, acc_sh.at[pl.ds(s * (NLOC // NS) + k * CHN, CHN)])

    plsc.subcore_barrier()

    ebase = s * EPT

    @pl.loop(0, NPH)
    def _(ph):
        pltpu.sync_copy(send_hbm.at[pl.ds(ebase + ph * PH, PH)], rawS)
        pltpu.sync_copy(recv_hbm.at[pl.ds(ebase + ph * PH, PH)], rawR)

        def scan_body(g, cnt):
            off = g * 16
            sv = rawS[pl.ds(off, 16)]
            rv = rawR[pl.ds(off, 16)]
            rloc = rv - base
            m = jnp.logical_and(rloc >= 0, rloc < NLOC)
            packed = jnp.bitwise_or(sv, lax.shift_left(rloc, SBITS))
            inc = plsc.cumsum(m.astype(jnp.int32))
            plsc.store_scatter(cpk, [cnt + inc - 1], packed, mask=m)
            return cnt + jnp.sum(m.astype(jnp.int32))

        cnt = lax.fori_loop(0, PH // 16, scan_body, jnp.int32(0))

        pad_pk = jnp.full((16,), PAD_PK, jnp.int32)
        for t in range(PAIR // 16):
            plsc.store_scatter(cpk, [cnt + iota16 + t * 16], pad_pk)
        npairs = (cnt + PAIR - 1) // PAIR

        def pair_body(k, carry):
            o = k * PAIR
            pA = jnp.bitwise_and(k, 1)
            pB = 1 - pA

            @pl.when(k > 0)
            def _():
                pltpu.make_async_copy(
                    hbufA, acc_sh.at[rstage.at[pA]], ssA).wait()
                pltpu.make_async_copy(
                    hbufB, acc_sh.at[rstage.at[pB]], ssB).wait()

            for q, pr in ((0, pA), (1, pB)):
                for t in range(CHN // 16):
                    w = cpk[pl.ds(o + q * CHN + t * 16, 16)]
                    sstage[pr, pl.ds(t * 16, 16)] = jnp.bitwise_and(w, mask_s)
                    rstage[pr, pl.ds(t * 16, 16)] = lax.shift_right_logical(
                        w, SBITS)
            gA = pltpu.async_copy(hs_hbm.at[sstage.at[pA]], hbufA, gsA)
            gB = pltpu.async_copy(hs_hbm.at[sstage.at[pB]], hbufB, gsB)
            gA.wait()
            pltpu.async_copy(hbufA, acc_sh.at[rstage.at[pA]], ssA, add=True)
            gB.wait()
            pltpu.async_copy(hbufB, acc_sh.at[rstage.at[pB]], ssB, add=True)
            return carry

        lax.fori_loop(0, npairs, pair_body, jnp.int32(0))

        @pl.when(npairs > 0)
        def _():
            pL = jnp.bitwise_and(npairs - 1, 1)
            pltpu.make_async_copy(hbufA, acc_sh.at[rstage.at[pL]],
                                  ssA).wait()
            pltpu.make_async_copy(hbufB, acc_sh.at[rstage.at[1 - pL]],
                                  ssB).wait()

    plsc.subcore_barrier()
    pltpu.sync_copy(acc_sh.at[pl.ds(s * (NLOC // NS), NLOC // NS)],
                    acc_out.at[pl.ds(base + s * (NLOC // NS), NLOC // NS)])


CPW = N_CHUNKS // (NC * NS)


@functools.partial(
    pl.kernel,
    out_type=jax.ShapeDtypeStruct((NC, N_PAD, GW), jnp.float32),
    mesh=_mesh,
    scratch_types=[
        pltpu.VMEM_SHARED((N_PAD, GW), jnp.float32),
        pltpu.VMEM((BLK, CH), jnp.int32),
        pltpu.VMEM((BLK, CH), jnp.int32),
        pltpu.VMEM((2, CH), jnp.float32),
        pltpu.VMEM((2, DE, CH), jnp.float32),
        pltpu.VMEM((CH, GW), jnp.float32),
        pltpu.VMEM((CH, GW), jnp.float32),
        pltpu.SemaphoreType.DMA,
        pltpu.SemaphoreType.DMA,
        pltpu.SemaphoreType.DMA,
        pltpu.SemaphoreType.DMA,
    ],
    compiler_params=_sc_params,
)
def _sc_gfeat(send_hbm, recv_hbm, eft_hbm, rs_hbm,
              g_out, g_sh, sidx, ridx, rsbuf, eftv, gbufA, gbufB,
              leA, leB, ssA, ssB):
    c = lax.axis_index("c")
    s = lax.axis_index("s")
    w = s * NC + c
    zeros16 = jnp.zeros((16,), jnp.float32)
    iota16 = lax.iota(jnp.int32, 16)
    gbufs = (gbufA, gbufB)
    lsems = (leA, leB)
    ssems = (ssA, ssB)

    @pl.loop(0, CH)
    def _(i):
        @pl.loop(0, GW, step=16)
        def _(q):
            gbufA[i, pl.ds(q, 16)] = zeros16
            gbufB[i, pl.ds(q, 16)] = zeros16

    @pl.loop(0, ROWS // CH)
    def _(k):
        pltpu.sync_copy(gbufA, g_sh.at[pl.ds(s * ROWS + k * CH, CH)])

    plsc.subcore_barrier()

    @pl.loop(0, CPW // BLK)
    def _(b):
        base = w * CPW + b * BLK
        pltpu.sync_copy(send_hbm.at[pl.ds(base, BLK)], sidx)
        pltpu.sync_copy(recv_hbm.at[pl.ds(base, BLK)], ridx)

        def load(j, p):
            e = pltpu.async_copy(
                eft_hbm.at[:, pl.ds((base + j) * CH, CH)], eftv.at[p],
                lsems[p])
            r = pltpu.async_copy(rs_hbm.at[sidx.at[j]], rsbuf.at[p],
                                 lsems[p])
            return e, r

        ld = {0: load(0, 0), 1: load(1, 1)}
        sc = {}
        for j in range(BLK):
            p = j & 1
            ld[j][0].wait()
            ld[j][1].wait()
            if j >= 2:
                sc[j - 2].wait()
            for g in range(CH // 16):
                rg = rsbuf[p, pl.ds(g * 16, 16)]
                rows = iota16 + g * 16
                for d in range(DE):
                    v = eftv[p, d, pl.ds(g * 16, 16)] * rg
                    plsc.store_scatter(
                        gbufs[p], [rows, jnp.full((16,), d, jnp.int32)], v)
                plsc.store_scatter(
                    gbufs[p], [rows, jnp.full((16,), DE, jnp.int32)], rg)
            sc[j] = pltpu.async_copy(
                gbufs[p], g_sh.at[ridx.at[j]], ssems[p], add=True)
            if j + 2 < BLK:
                ld[j + 2] = load(j + 2, p)
        sc[BLK - 2].wait()
        sc[BLK - 1].wait()

    plsc.subcore_barrier()
    pltpu.sync_copy(g_sh.at[pl.ds(s * ROWS, ROWS)],
                    g_out.at[c].at[pl.ds(s * ROWS, ROWS)])


def _tc_h_body(x_ref, w_ref, b_ref, h_ref):
    h_ref[...] = (
        jax.lax.dot_general(x_ref[...], w_ref[...], (((1,), (0,)), ((), ())),
                            precision=jax.lax.Precision.HIGHEST,
                            preferred_element_type=jnp.float32)
        + b_ref[...])


def _tc_scale_body(h_ref, hist_ref, hs_ref, rs_ref):
    i = pl.program_id(0)
    deg = jnp.sum(hist_ref[...], axis=(0, 2)) + 1.0
    rs = lax.rsqrt(deg)
    rows = i * h_ref.shape[0] + lax.broadcasted_iota(
        jnp.int32, (h_ref.shape[0], 1), 0)
    hs_ref[...] = jnp.where(rows < N, h_ref[...] * rs[:, None], 0.0)
    rs_ref[...] = rs


def _tc_final_body(acc_ref, g_ref, hs_ref, rs_ref, we_ref, be_ref, out_ref):
    g = g_ref[0, ...] + g_ref[1, ...]
    ef_part = jax.lax.dot_general(
        g[:, :DE], we_ref[...], (((1,), (0,)), ((), ())),
        precision=jax.lax.Precision.HIGHEST,
        preferred_element_type=jnp.float32)
    bias_part = g[:, DE:DE + 1] * be_ref[...][None, :]
    out_ref[...] = rs_ref[...][:, None] * (
        hs_ref[...] + acc_ref[...] + ef_part + bias_part)


@jax.jit
def kernel(node_features, senders, receivers, edge_features,
           W_kernel, W_bias, We_kernel, We_bias):
    senders = senders.astype(jnp.int32)
    receivers = receivers.astype(jnp.int32)

    pad_e = E_PAD - E
    s_flat = jnp.concatenate([senders, jnp.full((pad_e,), N, jnp.int32)])
    r_flat = jnp.concatenate([receivers, jnp.full((pad_e,), N, jnp.int32)])
    s2d = s_flat.reshape(N_CHUNKS, CH)
    r2d = r_flat.reshape(N_CHUNKS, CH)
    eft = jnp.concatenate(
        [edge_features.T, jnp.zeros((DE, pad_e), jnp.float32)], axis=1)
    xpad = jnp.concatenate(
        [node_features, jnp.zeros((N_PAD - N, D), jnp.float32)])

    hist = _sc_hist(r2d)

    RB = 1024
    grid = (N_PAD // RB,)

    h = pl.pallas_call(
        _tc_h_body,
        grid=grid,
        in_specs=[pl.BlockSpec((RB, D), lambda i: (i, 0)),
                  pl.BlockSpec((D, D), lambda i: (0, 0)),
                  pl.BlockSpec((D,), lambda i: (0,))],
        out_specs=pl.BlockSpec((RB, D), lambda i: (i, 0)),
        out_shape=jax.ShapeDtypeStruct((N_PAD, D), jnp.float32),
    )(xpad, W_kernel, W_bias)

    hs, rs = pl.pallas_call(
        _tc_scale_body,
        grid=grid,
        in_specs=[pl.BlockSpec((RB, D), lambda i: (i, 0)),
                  pl.BlockSpec((NC, RB, 16), lambda i: (0, i, 0))],
        out_specs=(pl.BlockSpec((RB, D), lambda i: (i, 0)),
                   pl.BlockSpec((RB,), lambda i: (i,))),
        out_shape=(jax.ShapeDtypeStruct((N_PAD, D), jnp.float32),
                   jax.ShapeDtypeStruct((N_PAD,), jnp.float32)),
    )(h, hist)

    acc = _sc_edges(hs.reshape(N_PAD, 2, D // 2),
                    s_flat, r_flat).reshape(N_PAD, D)
    g2 = _sc_gfeat(s2d, r2d, eft, rs)

    out = pl.pallas_call(
        _tc_final_body,
        grid=grid,
        in_specs=[pl.BlockSpec((RB, D), lambda i: (i, 0)),
                  pl.BlockSpec((NC, RB, GW), lambda i: (0, i, 0)),
                  pl.BlockSpec((RB, D), lambda i: (i, 0)),
                  pl.BlockSpec((RB,), lambda i: (i,)),
                  pl.BlockSpec((DE, D), lambda i: (0, 0)),
                  pl.BlockSpec((D,), lambda i: (0,))],
        out_specs=pl.BlockSpec((RB, D), lambda i: (i, 0)),
        out_shape=jax.ShapeDtypeStruct((N_PAD, D), jnp.float32),
    )(acc, g2, hs, rs, We_kernel, We_bias)

    return out[:N]


# speedup vs baseline: 2.2203x; 2.1929x over previous
import dataclasses
import functools
import jax
import jax.numpy as jnp
from jax import lax
from jax.experimental import pallas as pl
from jax.experimental.pallas import tpu as pltpu
from jax.experimental.pallas import tpu_sc as plsc

N = 10000
E = 160000
D = 256
DE = 16

NC = 2
NS = 16
CH = 128
N_PAD = 10240
E_PAD = 163840
N_CHUNKS = E_PAD // CH
CPS = N_CHUNKS // NS
ROWS = N_PAD // NS
HALF = D // 2
GW = 32
BLK = 8

_mesh = plsc.VectorSubcoreMesh(core_axis_name="c", subcore_axis_name="s",
                               num_cores=NC, num_subcores=NS)

_sc_params = pltpu.CompilerParams()
if "needs_layout_passes" in pltpu.CompilerParams.__dataclass_fields__:
    _sc_params = dataclasses.replace(_sc_params, needs_layout_passes=False)


@functools.partial(
    pl.kernel,
    out_type=jax.ShapeDtypeStruct((NC, N_PAD, 16), jnp.float32),
    mesh=_mesh,
    scratch_types=[
        pltpu.VMEM_SHARED((N_PAD, 16), jnp.float32),
        pltpu.VMEM((CPS // 2, CH), jnp.int32),
        pltpu.VMEM((CH, 16), jnp.float32),
        pltpu.VMEM((CH, 16), jnp.float32),
    ],
)
def _sc_hist(recv_hbm, hist_out, hist_sh, ridx, onehot, zbuf):
    c = lax.axis_index("c")
    s = lax.axis_index("s")
    w = s * NC + c
    zeros16 = jnp.zeros((16,), jnp.float32)
    one0 = jnp.where(lax.iota(jnp.int32, 16) == 0,
                     jnp.float32(1.0), jnp.float32(0.0))

    @pl.loop(0, CH)
    def _(i):
        onehot[i, :] = one0
        zbuf[i, :] = zeros16

    @pl.loop(0, ROWS // CH)
    def _(k):
        pltpu.sync_copy(zbuf, hist_sh.at[pl.ds(s * ROWS + k * CH, CH)])

    pltpu.sync_copy(recv_hbm.at[pl.ds(w * (CPS // 2), CPS // 2)], ridx)
    plsc.subcore_barrier()

    @pl.loop(0, CPS // 2)
    def _(j):
        pltpu.sync_copy(onehot, hist_sh.at[ridx.at[j]], add=True)

    plsc.subcore_barrier()
    pltpu.sync_copy(hist_sh.at[pl.ds(s * ROWS, ROWS)],
                    hist_out.at[c].at[pl.ds(s * ROWS, ROWS)])


@functools.partial(
    pl.kernel,
    out_type=jax.ShapeDtypeStruct((NC, N_PAD, HALF), jnp.float32),
    mesh=_mesh,
    scratch_types=[
        pltpu.VMEM_SHARED((N_PAD, HALF), jnp.float32),
        pltpu.VMEM((BLK, CH), jnp.int32),
        pltpu.VMEM((BLK, CH), jnp.int32),
        pltpu.VMEM((CH, HALF), jnp.float32),
        pltpu.VMEM((CH, HALF), jnp.float32),
        pltpu.SemaphoreType.DMA,
        pltpu.SemaphoreType.DMA,
        pltpu.SemaphoreType.DMA,
        pltpu.SemaphoreType.DMA,
    ],
    compiler_params=_sc_params,
)
def _sc_edges(hs_hbm, send_hbm, recv_hbm,
              acc_out, acc_sh, sidx, ridx, hbufA, hbufB,
              gsA, gsB, ssA, ssB):
    c = lax.axis_index("c")
    s = lax.axis_index("s")
    zeros16 = jnp.zeros((16,), jnp.float32)
    bufs = (hbufA, hbufB)
    gsems = (gsA, gsB)
    ssems = (ssA, ssB)

    @pl.loop(0, CH)
    def _(i):
        @pl.loop(0, HALF, step=16)
        def _(q):
            hbufA[i, pl.ds(q, 16)] = zeros16

    @pl.loop(0, ROWS // CH)
    def _(k):
        pltpu.sync_copy(hbufA, acc_sh.at[pl.ds(s * ROWS + k * CH, CH)])

    plsc.subcore_barrier()

    @pl.loop(0, CPS // BLK)
    def _(b):
        pltpu.sync_copy(send_hbm.at[pl.ds(s * CPS + b * BLK, BLK)], sidx)
        pltpu.sync_copy(recv_hbm.at[pl.ds(s * CPS + b * BLK, BLK)], ridx)

        g = {}
        sc = {}
        g[0] = pltpu.async_copy(hs_hbm.at[c].at[sidx.at[0]], bufs[0], gsA)
        g[1] = pltpu.async_copy(hs_hbm.at[c].at[sidx.at[1]], bufs[1], gsB)
        for j in range(BLK):
            p = j & 1
            g[j].wait()
            sc[j] = pltpu.async_copy(
                bufs[p], acc_sh.at[ridx.at[j]], ssems[p], add=True)
            if j + 2 < BLK:
                sc[j].wait()
                g[j + 2] = pltpu.async_copy(
                    hs_hbm.at[c].at[sidx.at[j + 2]], bufs[p], gsems[p])
        sc[BLK - 2].wait()
        sc[BLK - 1].wait()

    plsc.subcore_barrier()
    pltpu.sync_copy(acc_sh.at[pl.ds(s * ROWS, ROWS)],
                    acc_out.at[c].at[pl.ds(s * ROWS, ROWS)])


CPW = N_CHUNKS // (NC * NS)


@functools.partial(
    pl.kernel,
    out_type=jax.ShapeDtypeStruct((NC, N_PAD, GW), jnp.float32),
    mesh=_mesh,
    scratch_types=[
        pltpu.VMEM_SHARED((N_PAD, GW), jnp.float32),
        pltpu.VMEM((BLK, CH), jnp.int32),
        pltpu.VMEM((BLK, CH), jnp.int32),
        pltpu.VMEM((N_PAD,), jnp.float32),
        pltpu.VMEM((2, DE, CH), jnp.float32),
        pltpu.VMEM((CH, GW), jnp.float32),
        pltpu.VMEM((CH, GW), jnp.float32),
        pltpu.SemaphoreType.DMA,
        pltpu.SemaphoreType.DMA,
        pltpu.SemaphoreType.DMA,
        pltpu.SemaphoreType.DMA,
    ],
    compiler_params=_sc_params,
)
def _sc_gfeat(send_hbm, recv_hbm, eft_hbm, rs_hbm,
              g_out, g_sh, sidx, ridx, rs_v, eftv, gbufA, gbufB,
              leA, leB, ssA, ssB):
    c = lax.axis_index("c")
    s = lax.axis_index("s")
    w = s * NC + c
    zeros16 = jnp.zeros((16,), jnp.float32)
    iota16 = lax.iota(jnp.int32, 16)
    gbufs = (gbufA, gbufB)
    lsems = (leA, leB)
    ssems = (ssA, ssB)

    @pl.loop(0, CH)
    def _(i):
        @pl.loop(0, GW, step=16)
        def _(q):
            gbufA[i, pl.ds(q, 16)] = zeros16
            gbufB[i, pl.ds(q, 16)] = zeros16

    @pl.loop(0, ROWS // CH)
    def _(k):
        pltpu.sync_copy(gbufA, g_sh.at[pl.ds(s * ROWS + k * CH, CH)])

    pltpu.sync_copy(rs_hbm, rs_v)
    plsc.subcore_barrier()

    @pl.loop(0, CPW // BLK)
    def _(b):
        base = w * CPW + b * BLK
        pltpu.sync_copy(send_hbm.at[pl.ds(base, BLK)], sidx)
        pltpu.sync_copy(recv_hbm.at[pl.ds(base, BLK)], ridx)

        def load(j, p):
            return pltpu.async_copy(
                eft_hbm.at[:, pl.ds((base + j) * CH, CH)], eftv.at[p],
                lsems[p])

        ld = {0: load(0, 0), 1: load(1, 1)}
        sc = {}
        for j in range(BLK):
            p = j & 1
            ld[j].wait()
            if j >= 2:
                sc[j - 2].wait()
            for g in range(CH // 16):
                s16 = sidx[j, pl.ds(g * 16, 16)]
                rg = plsc.load_gather(rs_v, [s16])
                rows = iota16 + g * 16
                for d in range(DE):
                    v = eftv[p, d, pl.ds(g * 16, 16)] * rg
                    plsc.store_scatter(
                        gbufs[p], [rows, jnp.full((16,), d, jnp.int32)], v)
                plsc.store_scatter(
                    gbufs[p], [rows, jnp.full((16,), DE, jnp.int32)], rg)
            sc[j] = pltpu.async_copy(
                gbufs[p], g_sh.at[ridx.at[j]], ssems[p], add=True)
            if j + 2 < BLK:
                ld[j + 2] = load(j + 2, p)
        sc[BLK - 2].wait()
        sc[BLK - 1].wait()

    plsc.subcore_barrier()
    pltpu.sync_copy(g_sh.at[pl.ds(s * ROWS, ROWS)],
                    g_out.at[c].at[pl.ds(s * ROWS, ROWS)])


def _tc_h_body(x_ref, w_ref, b_ref, h_ref):
    h_ref[...] = (
        jax.lax.dot_general(x_ref[...], w_ref[...], (((1,), (0,)), ((), ())),
                            precision=jax.lax.Precision.HIGHEST,
                            preferred_element_type=jnp.float32)
        + b_ref[...])


def _tc_scale_body(h_ref, hist_ref, hs_ref, rs_ref):
    deg = jnp.sum(hist_ref[...], axis=(0, 2)) + 1.0
    rs = lax.rsqrt(deg)
    hs = h_ref[...] * rs[:, None]
    hs_ref[0, ...] = hs[:, :HALF]
    hs_ref[1, ...] = hs[:, HALF:]
    rs_ref[...] = rs


def _tc_final_body(acc_ref, g_ref, hs_ref, rs_ref, we_ref, be_ref, out_ref):
    g = g_ref[0, ...] + g_ref[1, ...]
    ef_part = jax.lax.dot_general(
        g[:, :DE], we_ref[...], (((1,), (0,)), ((), ())),
        precision=jax.lax.Precision.HIGHEST,
        preferred_element_type=jnp.float32)
    hs = jnp.concatenate([hs_ref[0, ...], hs_ref[1, ...]], axis=1)
    acc = jnp.concatenate([acc_ref[0, ...], acc_ref[1, ...]], axis=1)
    bias_part = g[:, DE:DE + 1] * be_ref[...][None, :]
    out_ref[...] = rs_ref[...][:, None] * (hs + acc + ef_part + bias_part)


@jax.jit
def kernel(node_features, senders, receivers, edge_features,
           W_kernel, W_bias, We_kernel, We_bias):
    senders = senders.astype(jnp.int32)
    receivers = receivers.astype(jnp.int32)

    pad_e = E_PAD - E
    s2d = jnp.concatenate(
        [senders, jnp.full((pad_e,), N, jnp.int32)]).reshape(N_CHUNKS, CH)
    r2d = jnp.concatenate(
        [receivers, jnp.full((pad_e,), N, jnp.int32)]).reshape(N_CHUNKS, CH)
    eft = jnp.concatenate(
        [edge_features.T, jnp.zeros((DE, pad_e), jnp.float32)], axis=1)
    xpad = jnp.concatenate(
        [node_features, jnp.zeros((N_PAD - N, D), jnp.float32)])

    hist = _sc_hist(r2d)

    RB = 1024
    grid = (N_PAD // RB,)

    h = pl.pallas_call(
        _tc_h_body,
        grid=grid,
        in_specs=[pl.BlockSpec((RB, D), lambda i: (i, 0)),
                  pl.BlockSpec((D, D), lambda i: (0, 0)),
                  pl.BlockSpec((D,), lambda i: (0,))],
        out_specs=pl.BlockSpec((RB, D), lambda i: (i, 0)),
        out_shape=jax.ShapeDtypeStruct((N_PAD, D), jnp.float32),
    )(xpad, W_kernel, W_bias)

    hs2, rs = pl.pallas_call(
        _tc_scale_body,
        grid=grid,
        in_specs=[pl.BlockSpec((RB, D), lambda i: (i, 0)),
                  pl.BlockSpec((NC, RB, 16), lambda i: (0, i, 0))],
        out_specs=(pl.BlockSpec((NC, RB, HALF), lambda i: (0, i, 0)),
                   pl.BlockSpec((RB,), lambda i: (i,))),
        out_shape=(jax.ShapeDtypeStruct((NC, N_PAD, HALF), jnp.float32),
                   jax.ShapeDtypeStruct((N_PAD,), jnp.float32)),
    )(h, hist)

    acc2 = _sc_edges(hs2, s2d, r2d)
    g2 = _sc_gfeat(s2d, r2d, eft, rs)

    out = pl.pallas_call(
        _tc_final_body,
        grid=grid,
        in_specs=[pl.BlockSpec((NC, RB, HALF), lambda i: (0, i, 0)),
                  pl.BlockSpec((NC, RB, GW), lambda i: (0, i, 0)),
                  pl.BlockSpec((NC, RB, HALF), lambda i: (0, i, 0)),
                  pl.BlockSpec((RB,), lambda i: (i,)),
                  pl.BlockSpec((DE, D), lambda i: (0, 0)),
                  pl.BlockSpec((D,), lambda i: (0,))],
        out_specs=pl.BlockSpec((RB, D), lambda i: (i, 0)),
        out_shape=jax.ShapeDtypeStruct((N_PAD, D), jnp.float32),
    )(acc2, g2, hs2, rs, We_kernel, We_bias)

    return out[:N]
